# baseline (speedup 1.0000x reference)
import functools

import jax
import jax.numpy as jnp
from jax import lax
from jax.experimental import pallas as pl
from jax.experimental.pallas import tpu as pltpu
from jax.experimental.pallas import tpu_sc as plsc

C = 512
HW = 4096
K = 4096
TOPK = 819
KT = 512
NKT = K // KT
PT = 512
NPT = HW // PT
EPS = 1e-8


def _topk_filter(r):
    bits = lax.bitcast_convert_type(r, jnp.int32)

    def bs(_, lohi):
        lo, hi = lohi
        mid = lo + ((hi - lo + 1) >> 1)
        cnt = jnp.sum((bits >= mid).astype(jnp.int32))
        big = cnt >= TOPK
        return (jnp.where(big, mid, lo), jnp.where(big, hi, mid - 1))

    thr, _ = lax.fori_loop(
        0, 31, bs, (jnp.int32(0), jnp.int32(0x3F7FFFFF)))
    gt = bits > thr
    eq = bits == thr
    need = (TOPK - jnp.sum(gt.astype(jnp.int32))).astype(jnp.float32)
    eqf = eq.astype(jnp.float32)
    ia = lax.broadcasted_iota(jnp.int32, (128, 128), 0)
    ib = lax.broadcasted_iota(jnp.int32, (128, 128), 1)
    upper_incl = (ia <= ib).astype(jnp.float32)
    incl = jnp.dot(eqf, upper_incl, preferred_element_type=jnp.float32)
    tot = incl[:, 127:128]
    ra = lax.broadcasted_iota(jnp.int32, (32, 32), 0)
    rb = lax.broadcasted_iota(jnp.int32, (32, 32), 1)
    lower_strict = (rb < ra).astype(jnp.float32)
    offs = jnp.dot(lower_strict, tot, preferred_element_type=jnp.float32)
    excl = incl - eqf + offs
    keep = gt | (eq & (excl < need))
    return r * keep.astype(jnp.float32)


def _simstep_body(r_ref, cb_ref, a_ref, rf_ref, g_ref, max_ref, idx_ref):
    step = pl.program_id(0)

    @pl.when(step == 0)
    def _():
        rf_ref[...] = _topk_filter(r_ref[...])

    sim = jnp.dot(cb_ref[...], a_ref[...],
                  preferred_element_type=jnp.float32)
    ids = lax.broadcasted_iota(jnp.int32, (KT, HW), 0) + step * KT
    bmax = jnp.max(sim, axis=0, keepdims=True)
    bidx = jnp.min(jnp.where(sim == bmax, ids, K), axis=0, keepdims=True)

    @pl.when(step == 0)
    def _():
        max_ref[...] = bmax
        idx_ref[...] = bidx

    @pl.when(step > 0)
    def _():
        take = bmax > max_ref[...]
        idx_ref[...] = jnp.where(take, bidx, idx_ref[...])
        max_ref[...] = jnp.where(take, bmax, max_ref[...])

    @pl.when(step == NKT - 1)
    def _():
        g_ref[...] = idx_ref[...]


def _stage1(r2d, a_mat, cb, interpret=False):
    return pl.pallas_call(
        _simstep_body,
        grid=(NKT,),
        in_specs=[
            pl.BlockSpec((32, 128), lambda i: (0, 0)),
            pl.BlockSpec((KT, C), lambda i: (i, 0)),
            pl.BlockSpec((C, HW), lambda i: (0, 0)),
        ],
        out_specs=[
            pl.BlockSpec((32, 128), lambda i: (0, 0)),
            pl.BlockSpec((1, HW), lambda i: (0, 0)),
        ],
        out_shape=[
            jax.ShapeDtypeStruct((32, 128), jnp.float32),
            jax.ShapeDtypeStruct((1, HW), jnp.int32),
        ],
        scratch_shapes=[
            pltpu.VMEM((1, HW), jnp.float32),
            pltpu.VMEM((1, HW), jnp.int32),
        ],
        interpret=interpret,
    )(r2d, cb, a_mat)


@functools.cache
def _sc_gather_kernel():
    mesh = plsc.VectorSubcoreMesh(core_axis_name="c", subcore_axis_name="s")

    @functools.partial(
        pl.kernel,
        out_type=jax.ShapeDtypeStruct((HW, C), jnp.float32),
        mesh=mesh,
        scratch_types=[
            pltpu.VMEM((128,), jnp.int32),
            pltpu.VMEM((128, C), jnp.float32),
            pltpu.SemaphoreType.DMA,
        ],
    )
    def _sc_gather(idx_hbm, table_hbm, out_hbm, idx_v, rows_v, sem):
        wid = lax.axis_index("s") * 2 + lax.axis_index("c")
        base = wid * 128
        pltpu.sync_copy(idx_hbm.at[pl.ds(base, 128)], idx_v)
        pltpu.async_copy(table_hbm.at[idx_v], rows_v, sem).wait()
        pltpu.sync_copy(rows_v, out_hbm.at[pl.ds(base, 128)])

    return _sc_gather


def _final_body(a_ref, cbg_ref, rf_ref, o_ref):
    cbg_t = jnp.transpose(cbg_ref[...])
    prod = a_ref[...] * cbg_t
    denom = jnp.sum(prod, axis=0, keepdims=True)
    s = prod / (denom + EPS)
    o_ref[...] = s * rf_ref[...]


def _stage3(a_mat, cbg, rf_row, interpret=False):
    return pl.pallas_call(
        _final_body,
        grid=(NPT,),
        in_specs=[
            pl.BlockSpec((C, PT), lambda i: (0, i)),
            pl.BlockSpec((PT, C), lambda i: (i, 0)),
            pl.BlockSpec((1, PT), lambda i: (0, i)),
        ],
        out_specs=pl.BlockSpec((C, PT), lambda i: (0, i)),
        out_shape=jax.ShapeDtypeStruct((C, HW), jnp.float32),
        interpret=interpret,
    )(a_mat, cbg, rf_row)


def kernel(a, r, compare_bank):
    a_mat = a.reshape(C, HW)
    r2d = r.reshape(32, 128)
    rf, g = _stage1(r2d, a_mat, compare_bank)
    cbg = _sc_gather_kernel()(g.reshape(HW), compare_bank)
    out = _stage3(a_mat, cbg, rf.reshape(1, HW))
    return out.reshape(a.shape)
